# Initial kernel scaffold; baseline (speedup 1.0000x reference)
import jax
import jax.numpy as jnp
from jax.experimental import pallas as pl


def kernel(rgb, lidar, W_ln):
    raise NotImplementedError("write your pallas kernel here")


# speedup vs baseline: 216.8979x; 216.8979x over previous
import functools

import jax
import jax.numpy as jnp
from jax import lax
from jax.experimental import pallas as pl
from jax.experimental.pallas import tpu as pltpu

NUM_CLASSES = 4
R2 = 9.0
K_MAX = 512
NROW = 6
NEG_INF = float('-inf')
POS_INF = float('inf')


def _prep_body(rgb_ref, lid_ref, w_ref, sc_ref, px_ref, py_ref):
    n = rgb_ref.shape[1]
    for b in range(2):
        r3 = rgb_ref[3 * b:3 * b + 3, :]
        lg = jnp.dot(w_ref[...], r3, preferred_element_type=jnp.float32)
        l0 = lg[0:1, :]
        l1 = lg[1:2, :]
        l2 = lg[2:3, :]
        l3 = lg[3:4, :]
        val = jnp.maximum(jnp.maximum(l0, l1), jnp.maximum(l2, l3))
        ind = jnp.where(
            (l0 >= l1) & (l0 >= l2) & (l0 >= l3), 0,
            jnp.where((l1 >= l2) & (l1 >= l3), 1, jnp.where(l2 >= l3, 2, 3)),
        )
        px = lid_ref[2 * b:2 * b + 1, :]
        py = lid_ref[2 * b + 1:2 * b + 2, :]
        for c in range(1, NUM_CLASSES):
            row = 3 * b + (c - 1)
            sc_ref[row:row + 1, :] = jnp.where(ind == c, val, NEG_INF)
            px_ref[row:row + 1, :] = px
            py_ref[row:row + 1, :] = py


def _nms_body(sc_ref, px_ref, py_ref, cx_ref, cy_ref, s_ref):
    n = sc_ref.shape[1]
    s_ref[...] = sc_ref[...]
    cx_ref[...] = jnp.full((NROW, K_MAX), POS_INF, jnp.float32)
    cy_ref[...] = jnp.full((NROW, K_MAX), POS_INF, jnp.float32)

    lane_n = lax.broadcasted_iota(jnp.int32, (NROW, n), 1)
    lane_k = lax.broadcasted_iota(jnp.int32, (NROW, K_MAX), 1)
    px = px_ref[...]
    py = py_ref[...]

    def cond(carry):
        i, cont = carry
        return cont & (i < K_MAX)

    def body(carry):
        i, _ = carry
        s = s_ref[...]
        mx = jnp.max(s, axis=1, keepdims=True)
        alive = mx > NEG_INF
        eq = s == mx
        idx = jnp.min(jnp.where(eq, lane_n, n), axis=1, keepdims=True)
        onehot = lane_n == idx
        xi = jnp.max(jnp.where(onehot, px, NEG_INF), axis=1, keepdims=True)
        yi = jnp.max(jnp.where(onehot, py, NEG_INF), axis=1, keepdims=True)
        d2 = (px - xi) ** 2 + (py - yi) ** 2
        s_ref[...] = jnp.where((d2 < R2) & alive, NEG_INF, s)
        hit = (lane_k == i) & alive
        cx_ref[...] = jnp.where(hit, xi, cx_ref[...])
        cy_ref[...] = jnp.where(hit, yi, cy_ref[...])
        return i + 1, jnp.any(alive)

    lax.while_loop(cond, body, (jnp.int32(0), True))


def _assign_body(sc_ref, px_ref, py_ref, cxt_ref, cyt_ref, ox_ref, oy_ref,
                 ax_ref, ay_ref, ac_ref, chunk):
    n = sc_ref.shape[1]
    nchunks = n // chunk
    sub_k = lax.broadcasted_iota(jnp.int32, (K_MAX, chunk), 0)
    ax_ref[...] = jnp.zeros((K_MAX, 8), jnp.float32)
    ay_ref[...] = jnp.zeros((K_MAX, 8), jnp.float32)
    ac_ref[...] = jnp.zeros((K_MAX, 8), jnp.float32)
    for r in range(NROW):
        cxs = cxt_ref[:, r:r + 1]
        cys = cyt_ref[:, r:r + 1]

        def chunk_body(c, _, r=r, cxs=cxs, cys=cys):
            sl = pl.ds(c * chunk, chunk)
            pxc = px_ref[r:r + 1, sl]
            pyc = py_ref[r:r + 1, sl]
            scc = sc_ref[r:r + 1, sl]
            w = jnp.where(scc > NEG_INF, 1.0, 0.0)
            dx = pxc - cxs
            dy = pyc - cys
            d2 = dx * dx + dy * dy
            best = jnp.min(d2, axis=0, keepdims=True)
            besti = jnp.min(
                jnp.where(d2 == best, sub_k, K_MAX), axis=0, keepdims=True)
            ohw = jnp.where(sub_k == besti, w, 0.0)
            ax_ref[:, r:r + 1] += jnp.sum(ohw * pxc, axis=1, keepdims=True)
            ay_ref[:, r:r + 1] += jnp.sum(ohw * pyc, axis=1, keepdims=True)
            ac_ref[:, r:r + 1] += jnp.sum(ohw, axis=1, keepdims=True)
            return 0

        lax.fori_loop(0, nchunks, chunk_body, 0)
    cnt = jnp.maximum(ac_ref[:, :NROW], 1.0)
    ox_ref[...] = ax_ref[:, :NROW] / cnt
    oy_ref[...] = ay_ref[:, :NROW] / cnt


@functools.partial(jax.jit, static_argnames=())
def kernel(rgb, lidar, W_ln):
    B, _, H, W = rgb.shape
    n = H * W
    rgb2 = rgb.reshape(B * 3, n)
    lid2 = lidar[:, :2].reshape(B * 2, n)

    sc6, px6, py6 = pl.pallas_call(
        _prep_body,
        out_shape=[jax.ShapeDtypeStruct((NROW, n), jnp.float32)] * 3,
        in_specs=[pl.BlockSpec(memory_space=pltpu.VMEM)] * 3,
        out_specs=[pl.BlockSpec(memory_space=pltpu.VMEM)] * 3,
    )(rgb2, lid2, W_ln)

    cx, cy = pl.pallas_call(
        _nms_body,
        out_shape=[jax.ShapeDtypeStruct((NROW, K_MAX), jnp.float32)] * 2,
        scratch_shapes=[pltpu.VMEM((NROW, n), jnp.float32)],
    )(sc6, px6, py6)

    ox, oy = pl.pallas_call(
        functools.partial(_assign_body, chunk=512),
        out_shape=[jax.ShapeDtypeStruct((K_MAX, NROW), jnp.float32)] * 2,
        scratch_shapes=[pltpu.VMEM((K_MAX, 8), jnp.float32)] * 3,
    )(sc6, px6, py6, cx.T, cy.T)

    out = jnp.stack([ox.T.reshape(B, 3, K_MAX), oy.T.reshape(B, 3, K_MAX)],
                    axis=-1)
    return out


# speedup vs baseline: 229.0367x; 1.0560x over previous
import functools

import jax
import jax.numpy as jnp
from jax import lax
from jax.experimental import pallas as pl
from jax.experimental.pallas import tpu as pltpu

NUM_CLASSES = 4
R2 = 9.0
K_MAX = 512
NROW = 6
NEG_INF = float('-inf')
POS_INF = float('inf')


def _prep_body(rgb_ref, lid_ref, w_ref, sc_ref, px_ref, py_ref):
    n = rgb_ref.shape[1]
    for b in range(2):
        r3 = rgb_ref[3 * b:3 * b + 3, :]
        lg = jnp.dot(w_ref[...], r3, preferred_element_type=jnp.float32)
        l0 = lg[0:1, :]
        l1 = lg[1:2, :]
        l2 = lg[2:3, :]
        l3 = lg[3:4, :]
        val = jnp.maximum(jnp.maximum(l0, l1), jnp.maximum(l2, l3))
        ind = jnp.where(
            (l0 >= l1) & (l0 >= l2) & (l0 >= l3), 0,
            jnp.where((l1 >= l2) & (l1 >= l3), 1, jnp.where(l2 >= l3, 2, 3)),
        )
        px = lid_ref[2 * b:2 * b + 1, :]
        py = lid_ref[2 * b + 1:2 * b + 2, :]
        for c in range(1, NUM_CLASSES):
            row = 3 * b + (c - 1)
            sc_ref[row:row + 1, :] = jnp.where(ind == c, val, NEG_INF)
            px_ref[row:row + 1, :] = px
            py_ref[row:row + 1, :] = py


def _nms_body(sc_ref, px_ref, py_ref, cx_ref, cy_ref, kc_ref, s_ref):
    n = sc_ref.shape[1]
    s_ref[...] = sc_ref[...]
    cx_ref[...] = jnp.full((NROW, K_MAX), POS_INF, jnp.float32)
    cy_ref[...] = jnp.full((NROW, K_MAX), POS_INF, jnp.float32)

    lane_n = lax.broadcasted_iota(jnp.int32, (NROW, n), 1)
    lane_k = lax.broadcasted_iota(jnp.int32, (NROW, K_MAX), 1)
    px = px_ref[...]
    py = py_ref[...]

    def cond(carry):
        i, cont = carry
        return cont & (i < K_MAX)

    def body(carry):
        i, _ = carry
        s = s_ref[...]
        mx = jnp.max(s, axis=1, keepdims=True)
        alive = mx > NEG_INF
        eq = s == mx
        idx = jnp.min(jnp.where(eq, lane_n, n), axis=1, keepdims=True)
        onehot = lane_n == idx
        xi = jnp.max(jnp.where(onehot, px, NEG_INF), axis=1, keepdims=True)
        yi = jnp.max(jnp.where(onehot, py, NEG_INF), axis=1, keepdims=True)
        d2 = (px - xi) ** 2 + (py - yi) ** 2
        s_ref[...] = jnp.where((d2 < R2) & alive, NEG_INF, s)
        hit = (lane_k == i) & alive
        cx_ref[...] = jnp.where(hit, xi, cx_ref[...])
        cy_ref[...] = jnp.where(hit, yi, cy_ref[...])
        return i + 1, jnp.any(alive)

    i_f, cont_f = lax.while_loop(cond, body, (jnp.int32(0), True))
    kc_ref[0] = jnp.where(cont_f, i_f, i_f - 1)


def _assign_body(kc_ref, sc_ref, px_ref, py_ref, cxt_ref, cyt_ref,
                 ox_ref, oy_ref, ax_ref, ay_ref, ac_ref, chunk, cc_sz):
    n = sc_ref.shape[1]
    nchunks = n // chunk
    sub_c = lax.broadcasted_iota(jnp.int32, (cc_sz, chunk), 0)
    ax_ref[...] = jnp.zeros((K_MAX, 8), jnp.float32)
    ay_ref[...] = jnp.zeros((K_MAX, 8), jnp.float32)
    ac_ref[...] = jnp.zeros((K_MAX, 8), jnp.float32)
    kmax = kc_ref[0]
    nc = (kmax + cc_sz - 1) // cc_sz
    for r in range(NROW):

        def chunk_body(c, _, r=r):
            sl = pl.ds(c * chunk, chunk)
            pxc = px_ref[r:r + 1, sl]
            pyc = py_ref[r:r + 1, sl]
            scc = sc_ref[r:r + 1, sl]
            w = jnp.where(scc > NEG_INF, 1.0, 0.0)

            def cc_argmin(cc, carry, r=r, pxc=pxc, pyc=pyc):
                best, besti = carry
                csl = pl.ds(cc * cc_sz, cc_sz)
                dx = pxc - cxt_ref[csl, r:r + 1]
                dy = pyc - cyt_ref[csl, r:r + 1]
                d2 = dx * dx + dy * dy
                bmin = jnp.min(d2, axis=0, keepdims=True)
                bidx = jnp.min(jnp.where(d2 == bmin, sub_c, cc_sz),
                               axis=0, keepdims=True) + cc * cc_sz
                upd = bmin < best
                return (jnp.where(upd, bmin, best),
                        jnp.where(upd, bidx, besti))

            best0 = jnp.full((1, chunk), POS_INF, jnp.float32)
            besti0 = jnp.zeros((1, chunk), jnp.int32)
            _, besti = lax.fori_loop(0, nc, cc_argmin, (best0, besti0))

            def cc_sums(cc, _, r=r, pxc=pxc, pyc=pyc, w=w, besti=besti):
                csl = pl.ds(cc * cc_sz, cc_sz)
                ohw = jnp.where(sub_c + cc * cc_sz == besti, w, 0.0)
                ax_ref[csl, r:r + 1] += jnp.sum(ohw * pxc, axis=1,
                                                keepdims=True)
                ay_ref[csl, r:r + 1] += jnp.sum(ohw * pyc, axis=1,
                                                keepdims=True)
                ac_ref[csl, r:r + 1] += jnp.sum(ohw, axis=1, keepdims=True)
                return 0

            lax.fori_loop(0, nc, cc_sums, 0)
            return 0

        lax.fori_loop(0, nchunks, chunk_body, 0)
    cnt = jnp.maximum(ac_ref[:, :NROW], 1.0)
    ox_ref[...] = ax_ref[:, :NROW] / cnt
    oy_ref[...] = ay_ref[:, :NROW] / cnt


@functools.partial(jax.jit, static_argnames=())
def kernel(rgb, lidar, W_ln):
    B, _, H, W = rgb.shape
    n = H * W
    rgb2 = rgb.reshape(B * 3, n)
    lid2 = lidar[:, :2].reshape(B * 2, n)

    sc6, px6, py6 = pl.pallas_call(
        _prep_body,
        out_shape=[jax.ShapeDtypeStruct((NROW, n), jnp.float32)] * 3,
        in_specs=[pl.BlockSpec(memory_space=pltpu.VMEM)] * 3,
        out_specs=[pl.BlockSpec(memory_space=pltpu.VMEM)] * 3,
    )(rgb2, lid2, W_ln)

    cx, cy, kc = pl.pallas_call(
        _nms_body,
        out_shape=[
            jax.ShapeDtypeStruct((NROW, K_MAX), jnp.float32),
            jax.ShapeDtypeStruct((NROW, K_MAX), jnp.float32),
            jax.ShapeDtypeStruct((1,), jnp.int32),
        ],
        out_specs=[
            pl.BlockSpec(memory_space=pltpu.VMEM),
            pl.BlockSpec(memory_space=pltpu.VMEM),
            pl.BlockSpec(memory_space=pltpu.SMEM),
        ],
        scratch_shapes=[pltpu.VMEM((NROW, n), jnp.float32)],
    )(sc6, px6, py6)

    ox, oy = pl.pallas_call(
        functools.partial(_assign_body, chunk=512, cc_sz=64),
        out_shape=[jax.ShapeDtypeStruct((K_MAX, NROW), jnp.float32)] * 2,
        in_specs=[pl.BlockSpec(memory_space=pltpu.SMEM)]
        + [pl.BlockSpec(memory_space=pltpu.VMEM)] * 5,
        scratch_shapes=[pltpu.VMEM((K_MAX, 8), jnp.float32)] * 3,
    )(kc, sc6, px6, py6, cx.T, cy.T)

    out = jnp.stack([ox.T.reshape(B, 3, K_MAX), oy.T.reshape(B, 3, K_MAX)],
                    axis=-1)
    return out


# speedup vs baseline: 248.6870x; 1.0858x over previous
import functools

import jax
import jax.numpy as jnp
from jax import lax
from jax.experimental import pallas as pl
from jax.experimental.pallas import tpu as pltpu

NUM_CLASSES = 4
R2 = 9.0
K_MAX = 512
NROW = 6
NEG_INF = float('-inf')
POS_INF = float('inf')


def _prep_body(rgb_ref, lid_ref, w_ref, sc_ref, px_ref, py_ref):
    n = rgb_ref.shape[1]
    for b in range(2):
        r3 = rgb_ref[3 * b:3 * b + 3, :]
        lg = jnp.dot(w_ref[...], r3, preferred_element_type=jnp.float32)
        l0 = lg[0:1, :]
        l1 = lg[1:2, :]
        l2 = lg[2:3, :]
        l3 = lg[3:4, :]
        val = jnp.maximum(jnp.maximum(l0, l1), jnp.maximum(l2, l3))
        ind = jnp.where(
            (l0 >= l1) & (l0 >= l2) & (l0 >= l3), 0,
            jnp.where((l1 >= l2) & (l1 >= l3), 1, jnp.where(l2 >= l3, 2, 3)),
        )
        px = lid_ref[2 * b:2 * b + 1, :]
        py = lid_ref[2 * b + 1:2 * b + 2, :]
        for c in range(1, NUM_CLASSES):
            row = 3 * b + (c - 1)
            sc_ref[row:row + 1, :] = jnp.where(ind == c, val, NEG_INF)
            px_ref[row:row + 1, :] = px
            py_ref[row:row + 1, :] = py


def _nms_body(sc_ref, px_ref, py_ref, cx_ref, cy_ref, kc_ref, s_ref, pv_ref):
    n = sc_ref.shape[1]
    s_ref[...] = sc_ref[...]
    pv_ref[...] = jnp.zeros((8, 128), jnp.float32)
    cx_ref[...] = jnp.full((NROW, K_MAX), POS_INF, jnp.float32)
    cy_ref[...] = jnp.full((NROW, K_MAX), POS_INF, jnp.float32)

    lane_k = lax.broadcasted_iota(jnp.int32, (NROW, K_MAX), 1)
    px = px_ref[...]
    py = py_ref[...]

    def cond(carry):
        i, cont = carry
        return cont & (i < K_MAX)

    def body(carry):
        i, _ = carry
        pxi = pv_ref[0:NROW, 0:1]
        pyi = pv_ref[0:NROW, 1:2]
        palive = pv_ref[0:NROW, 2:3] > 0
        s = s_ref[...]
        d2 = (px - pxi) ** 2 + (py - pyi) ** 2
        s = jnp.where((d2 < R2) & palive, NEG_INF, s)
        s_ref[...] = s
        mx = jnp.max(s, axis=1, keepdims=True)
        alive = mx > NEG_INF
        eq = s == mx
        xi = jnp.max(jnp.where(eq, px, NEG_INF), axis=1, keepdims=True)
        yi = jnp.max(jnp.where(eq, py, NEG_INF), axis=1, keepdims=True)
        hit = (lane_k == i) & alive
        cx_ref[...] = jnp.where(hit, xi, cx_ref[...])
        cy_ref[...] = jnp.where(hit, yi, cy_ref[...])
        pv_ref[0:NROW, 0:1] = xi
        pv_ref[0:NROW, 1:2] = yi
        pv_ref[0:NROW, 2:3] = jnp.where(alive, 1.0, 0.0)
        return i + 1, jnp.any(alive)

    i_f, cont_f = lax.while_loop(cond, body, (jnp.int32(0), True))
    kc_ref[0] = jnp.where(cont_f, i_f, i_f - 1)


def _assign_body(kc_ref, sc_ref, px_ref, py_ref, cxt_ref, cyt_ref,
                 ox_ref, oy_ref, ax_ref, ay_ref, ac_ref, chunk, cc_sz):
    n = sc_ref.shape[1]
    nchunks = n // chunk
    sub_c = lax.broadcasted_iota(jnp.int32, (cc_sz, chunk), 0)
    ax_ref[...] = jnp.zeros((K_MAX, 8), jnp.float32)
    ay_ref[...] = jnp.zeros((K_MAX, 8), jnp.float32)
    ac_ref[...] = jnp.zeros((K_MAX, 8), jnp.float32)
    kmax = kc_ref[0]
    nc = (kmax + cc_sz - 1) // cc_sz
    for r in range(NROW):

        def chunk_body(c, _, r=r):
            sl = pl.ds(c * chunk, chunk)
            pxc = px_ref[r:r + 1, sl]
            pyc = py_ref[r:r + 1, sl]
            scc = sc_ref[r:r + 1, sl]
            w = jnp.where(scc > NEG_INF, 1.0, 0.0)

            def cc_argmin(cc, carry, r=r, pxc=pxc, pyc=pyc):
                best, besti = carry
                csl = pl.ds(cc * cc_sz, cc_sz)
                dx = pxc - cxt_ref[csl, r:r + 1]
                dy = pyc - cyt_ref[csl, r:r + 1]
                d2 = dx * dx + dy * dy
                bmin = jnp.min(d2, axis=0, keepdims=True)
                bidx = jnp.min(jnp.where(d2 == bmin, sub_c, cc_sz),
                               axis=0, keepdims=True) + cc * cc_sz
                upd = bmin < best
                return (jnp.where(upd, bmin, best),
                        jnp.where(upd, bidx, besti))

            best0 = jnp.full((1, chunk), POS_INF, jnp.float32)
            besti0 = jnp.zeros((1, chunk), jnp.int32)
            _, besti = lax.fori_loop(0, nc, cc_argmin, (best0, besti0))

            def cc_sums(cc, _, r=r, pxc=pxc, pyc=pyc, w=w, besti=besti):
                csl = pl.ds(cc * cc_sz, cc_sz)
                ohw = jnp.where(sub_c + cc * cc_sz == besti, w, 0.0)
                ax_ref[csl, r:r + 1] += jnp.sum(ohw * pxc, axis=1,
                                                keepdims=True)
                ay_ref[csl, r:r + 1] += jnp.sum(ohw * pyc, axis=1,
                                                keepdims=True)
                ac_ref[csl, r:r + 1] += jnp.sum(ohw, axis=1, keepdims=True)
                return 0

            lax.fori_loop(0, nc, cc_sums, 0)
            return 0

        lax.fori_loop(0, nchunks, chunk_body, 0)
    cnt = jnp.maximum(ac_ref[:, :NROW], 1.0)
    ox_ref[...] = ax_ref[:, :NROW] / cnt
    oy_ref[...] = ay_ref[:, :NROW] / cnt


@functools.partial(jax.jit, static_argnames=())
def kernel(rgb, lidar, W_ln):
    B, _, H, W = rgb.shape
    n = H * W
    rgb2 = rgb.reshape(B * 3, n)
    lid2 = lidar[:, :2].reshape(B * 2, n)

    sc6, px6, py6 = pl.pallas_call(
        _prep_body,
        out_shape=[jax.ShapeDtypeStruct((NROW, n), jnp.float32)] * 3,
        in_specs=[pl.BlockSpec(memory_space=pltpu.VMEM)] * 3,
        out_specs=[pl.BlockSpec(memory_space=pltpu.VMEM)] * 3,
    )(rgb2, lid2, W_ln)

    cx, cy, kc = pl.pallas_call(
        _nms_body,
        out_shape=[
            jax.ShapeDtypeStruct((NROW, K_MAX), jnp.float32),
            jax.ShapeDtypeStruct((NROW, K_MAX), jnp.float32),
            jax.ShapeDtypeStruct((1,), jnp.int32),
        ],
        out_specs=[
            pl.BlockSpec(memory_space=pltpu.VMEM),
            pl.BlockSpec(memory_space=pltpu.VMEM),
            pl.BlockSpec(memory_space=pltpu.SMEM),
        ],
        scratch_shapes=[pltpu.VMEM((NROW, n), jnp.float32),
                        pltpu.VMEM((8, 128), jnp.float32)],
    )(sc6, px6, py6)

    ox, oy = pl.pallas_call(
        functools.partial(_assign_body, chunk=512, cc_sz=64),
        out_shape=[jax.ShapeDtypeStruct((K_MAX, NROW), jnp.float32)] * 2,
        in_specs=[pl.BlockSpec(memory_space=pltpu.SMEM)]
        + [pl.BlockSpec(memory_space=pltpu.VMEM)] * 5,
        scratch_shapes=[pltpu.VMEM((K_MAX, 8), jnp.float32)] * 3,
    )(kc, sc6, px6, py6, cx.T, cy.T)

    out = jnp.stack([ox.T.reshape(B, 3, K_MAX), oy.T.reshape(B, 3, K_MAX)],
                    axis=-1)
    return out
